# Initial kernel scaffold; baseline (speedup 1.0000x reference)
import jax
import jax.numpy as jnp
from jax.experimental import pallas as pl


def kernel(x, a, y, logits_x, logits_a, mu_y, logvar_y, logits_n, logits_w):
    raise NotImplementedError("write your pallas kernel here")


# speedup vs baseline: 1.1155x; 1.1155x over previous
import functools

import jax
import jax.numpy as jnp
import numpy as np
from jax.experimental import pallas as pl

_ND_X = 38
_NK_X = 16
_NK_A = 5
_TRIL_R, _TRIL_C = np.tril_indices(_ND_X, -1)
_ND_A = len(_TRIL_R)
_LOG_2PI = float(np.log(2.0 * np.pi))
_MM_DTYPE = jnp.bfloat16


def _body(xr_ref, ar_ref, mxf_ref, maf_ref, y_ref, lxmm_ref, lamm_ref,
          lxk_ref, lak_ref, mu_ref, lv_ref, ln_ref, lw_ref, out_ref):
    f32 = jnp.float32
    bf16 = _MM_DTYPE
    b = xr_ref.shape[0]

    iota_x = jax.lax.broadcasted_iota(jnp.int32, xr_ref.shape, 1) % _NK_X
    oh_x = (xr_ref[...] == iota_x).astype(bf16)
    iota_a = jax.lax.broadcasted_iota(jnp.int32, ar_ref.shape, 1) % _NK_A
    oh_a = (ar_ref[...] == iota_a).astype(bf16)

    xs = lxk_ref[...]
    mx = jnp.max(xs, axis=0)
    lse_x = mx + jnp.log(jnp.sum(jnp.exp(xs - mx[None]), axis=0))
    asrc = lak_ref[...]
    ma = jnp.max(asrc, axis=0)
    lse_a = ma + jnp.log(jnp.sum(jnp.exp(asrc - ma[None]), axis=0))

    acc = jnp.dot(oh_a, lamm_ref[...], preferred_element_type=f32)
    acc += jnp.dot(oh_x, lxmm_ref[...], preferred_element_type=f32)
    acc -= jnp.dot(maf_ref[...], lse_a.astype(bf16), preferred_element_type=f32)
    acc -= jnp.dot(mxf_ref[...], lse_x, preferred_element_type=f32)

    yv = y_ref[...]
    mu = mu_ref[...]
    lv = lv_ref[...]
    gauss = -0.5 * ((yv - mu) ** 2 / jnp.exp(lv) + lv + _LOG_2PI)

    lw = lw_ref[...]
    mw = jnp.max(lw, axis=1, keepdims=True)
    logw = lw - (mw + jnp.log(jnp.sum(jnp.exp(lw - mw), axis=1, keepdims=True)))

    tot = acc + gauss + logw
    mt = jnp.max(tot, axis=1, keepdims=True)
    lse_tot = mt + jnp.log(jnp.sum(jnp.exp(tot - mt), axis=1, keepdims=True))

    nb = jnp.sum(mxf_ref[...], axis=1, keepdims=True).astype(jnp.int32) - 1
    nb = jnp.clip(nb, 0, _ND_X - 1)
    ln = ln_ref[...]
    mn = jnp.max(ln, axis=1, keepdims=True)
    lsn = ln - (mn + jnp.log(jnp.sum(jnp.exp(ln - mn), axis=1, keepdims=True)))
    ohn = (nb == jax.lax.broadcasted_iota(jnp.int32, (b, _ND_X), 1)).astype(f32)
    logs_c = jnp.sum(ohn * lsn, axis=1, keepdims=True)

    out_ref[...] = logs_c + lse_tot


@jax.jit
def kernel(x, a, y, logits_x, logits_a, mu_y, logvar_y, logits_n, logits_w):
    b = x.shape[0]
    nc = logits_w.shape[0]
    f32 = jnp.float32
    bf16 = _MM_DTYPE

    xm = x.astype(jnp.int32) - 1
    mask_x = xm > -1
    xv = jnp.where(mask_x, xm, _NK_X)
    a_flat = a[:, _TRIL_R, _TRIL_C].astype(jnp.int32)
    mask_a = mask_x[:, _TRIL_R] & mask_x[:, _TRIL_C]
    av = jnp.where(mask_a, a_flat, _NK_A)

    xr = jnp.repeat(xv, _NK_X, axis=1)
    ar = jnp.repeat(av, _NK_A, axis=1)
    mxf = mask_x.astype(f32)
    maf = mask_a.astype(bf16)

    lx_mm = logits_x.reshape(nc, _ND_X * _NK_X).T.astype(bf16)
    la_mm = logits_a.reshape(nc, _ND_A * _NK_A).T.astype(bf16)
    lx_k = logits_x.transpose(2, 1, 0)
    la_k = logits_a.transpose(2, 1, 0)

    out = pl.pallas_call(
        _body,
        out_shape=jax.ShapeDtypeStruct((b, 1), f32),
    )(xr, ar, mxf, maf, y.reshape(b, 1), lx_mm, la_mm, lx_k, la_k,
      mu_y.reshape(1, nc), logvar_y.reshape(1, nc),
      logits_n.reshape(1, _ND_X), logits_w.reshape(1, nc))
    return out.reshape(b)
